# baseline (speedup 1.0000x reference)
import functools

import jax
import jax.numpy as jnp
from jax.experimental import pallas as pl
from jax.experimental.pallas import tpu as pltpu

_TOKEN_BLOCK = 2048
_WBLK = 1


def _moe_body(n_exp, wblk, n_wsteps, n_tok_blocks, blk,
              w_ref, b_ref, x_ref, gw_ref, gb_ref, o_ref, sb_scr, p_scr):
    step = pl.program_id(0)
    d = x_ref.shape[1]

    @pl.when(step < n_wsteps)
    def _w_phase():
        for j in range(wblk):
            sb_scr[pl.ds(step * wblk + j, 1), :d] = (
                jnp.sum(w_ref[j], axis=1)[None, :])
            sb_scr[pl.ds(step * wblk + j, 1), d:] = b_ref[pl.ds(j, 1), 0, :]

    @pl.when(step < n_tok_blocks)
    def _g_phase():
        x = x_ref[...]
        logits = jax.lax.dot_general(
            x, gw_ref[...], (((1,), (1,)), ((), ())),
            preferred_element_type=jnp.float32,
        ) + gb_ref[...]
        g = jax.nn.softmax(logits, axis=-1)
        e_ids = jax.lax.broadcasted_iota(jnp.int32, g.shape, 1)
        i1 = jnp.argmax(g, axis=-1)
        oh1 = e_ids == i1[:, None]
        i2 = jnp.argmax(jnp.where(oh1, -1.0, g), axis=-1)
        oh2 = e_ids == i2[:, None]
        p = jnp.where(oh1 | oh2, g, 0.0)
        p_scr[:, pl.ds(step * blk, blk)] = p.T

    @pl.when(step >= n_wsteps)
    def _t_phase():
        t = step - n_wsteps
        pt = p_scr[:, pl.ds(t * blk, blk)]
        q = jax.lax.dot_general(
            pt, sb_scr[...], (((0,), (0,)), ((), ())),
            preferred_element_type=jnp.float32,
            precision=jax.lax.Precision.HIGHEST,
        )
        o_ref[...] = x_ref[...] * q[:, :d] + q[:, d:]


def kernel(x, gate_weight, gate_bias, experts_weights, experts_bias):
    n_tokens, d_in = x.shape
    n_exp, _, d_out = experts_weights.shape
    blk = _TOKEN_BLOCK
    wblk = _WBLK
    n_tok_blocks = n_tokens // blk
    n_wsteps = n_exp // wblk
    assert n_wsteps >= n_tok_blocks

    body = functools.partial(_moe_body, n_exp, wblk, n_wsteps, n_tok_blocks,
                             blk)
    out = pl.pallas_call(
        body,
        grid=(n_wsteps + n_tok_blocks,),
        in_specs=[
            pl.BlockSpec((wblk, d_in, d_out),
                         lambda i: (jnp.minimum(i, n_wsteps - 1), 0, 0)),
            pl.BlockSpec((wblk, 1, d_out),
                         lambda i: (jnp.minimum(i, n_wsteps - 1), 0, 0)),
            pl.BlockSpec((blk, d_in),
                         lambda i: (jnp.where(i < n_wsteps,
                                              jnp.minimum(i, n_tok_blocks - 1),
                                              i - n_wsteps), 0)),
            pl.BlockSpec((n_exp, d_in), lambda i: (0, 0)),
            pl.BlockSpec((1, n_exp), lambda i: (0, 0)),
        ],
        out_specs=pl.BlockSpec((blk, d_out),
                               lambda i: (jnp.maximum(i - n_wsteps, 0), 0)),
        out_shape=jax.ShapeDtypeStruct((n_tokens, d_out), jnp.float32),
        scratch_shapes=[pltpu.VMEM((n_exp, d_in + d_out), jnp.float32),
                        pltpu.VMEM((n_exp, n_tokens), jnp.float32)],
    )(experts_weights, experts_bias.reshape(n_exp, 1, d_out), x, gate_weight,
      gate_bias.reshape(1, n_exp))
    return out


# speedup vs baseline: 1.0803x; 1.0803x over previous
import functools

import jax
import jax.numpy as jnp
from jax.experimental import pallas as pl
from jax.experimental.pallas import tpu as pltpu

_TOKEN_BLOCK = 1024
_WBLK = 2


def _moe_body(n_exp, wblk, n_wsteps, n_tok_blocks, blk,
              w_ref, b_ref, x_ref, gw_ref, gb_ref, o_ref, sb_scr, p_scr):
    step = pl.program_id(0)
    d = x_ref.shape[1]

    @pl.when(step < n_wsteps)
    def _w_phase():
        for j in range(wblk):
            sb_scr[pl.ds(step * wblk + j, 1), :d] = (
                jnp.sum(w_ref[j], axis=1)[None, :])
            sb_scr[pl.ds(step * wblk + j, 1), d:] = b_ref[pl.ds(j, 1), 0, :]

    @pl.when(step < n_tok_blocks)
    def _g_phase():
        x = x_ref[...]
        logits = jax.lax.dot_general(
            x, gw_ref[...], (((1,), (1,)), ((), ())),
            preferred_element_type=jnp.float32,
        ) + gb_ref[...]
        g = jax.nn.softmax(logits, axis=-1)
        e_ids = jax.lax.broadcasted_iota(jnp.int32, g.shape, 1)
        i1 = jnp.argmax(g, axis=-1)
        oh1 = e_ids == i1[:, None]
        i2 = jnp.argmax(jnp.where(oh1, -1.0, g), axis=-1)
        oh2 = e_ids == i2[:, None]
        p = jnp.where(oh1 | oh2, g, 0.0)
        p_scr[:, pl.ds(step * blk, blk)] = p.T

    @pl.when(step >= n_wsteps)
    def _t_phase():
        t = step - n_wsteps
        pt = p_scr[:, pl.ds(t * blk, blk)]
        q = jax.lax.dot_general(
            pt, sb_scr[...], (((0,), (0,)), ((), ())),
            preferred_element_type=jnp.float32,
            precision=jax.lax.Precision.HIGHEST,
        )
        o_ref[...] = x_ref[...] * q[:, :d] + q[:, d:]


def kernel(x, gate_weight, gate_bias, experts_weights, experts_bias):
    n_tokens, d_in = x.shape
    n_exp, _, d_out = experts_weights.shape
    blk = _TOKEN_BLOCK
    wblk = _WBLK
    n_tok_blocks = n_tokens // blk
    n_wsteps = n_exp // wblk
    assert n_wsteps >= n_tok_blocks

    body = functools.partial(_moe_body, n_exp, wblk, n_wsteps, n_tok_blocks,
                             blk)
    out = pl.pallas_call(
        body,
        grid=(n_wsteps + n_tok_blocks,),
        in_specs=[
            pl.BlockSpec((wblk, d_in, d_out),
                         lambda i: (jnp.minimum(i, n_wsteps - 1), 0, 0)),
            pl.BlockSpec((wblk, 1, d_out),
                         lambda i: (jnp.minimum(i, n_wsteps - 1), 0, 0)),
            pl.BlockSpec((blk, d_in),
                         lambda i: (jnp.where(i < n_wsteps,
                                              jnp.minimum(i, n_tok_blocks - 1),
                                              i - n_wsteps), 0)),
            pl.BlockSpec((n_exp, d_in), lambda i: (0, 0)),
            pl.BlockSpec((1, n_exp), lambda i: (0, 0)),
        ],
        out_specs=pl.BlockSpec((blk, d_out),
                               lambda i: (jnp.maximum(i - n_wsteps, 0), 0)),
        out_shape=jax.ShapeDtypeStruct((n_tokens, d_out), jnp.float32),
        scratch_shapes=[pltpu.VMEM((n_exp, d_in + d_out), jnp.float32),
                        pltpu.VMEM((n_exp, n_tokens), jnp.float32)],
    )(experts_weights, experts_bias.reshape(n_exp, 1, d_out), x, gate_weight,
      gate_bias.reshape(1, n_exp))
    return out


# speedup vs baseline: 1.1184x; 1.0353x over previous
import functools

import jax
import jax.numpy as jnp
from jax.experimental import pallas as pl
from jax.experimental.pallas import tpu as pltpu

_TOKEN_BLOCK = 1024
_WBLK = 2


def _moe_body(n_exp, wblk, n_wsteps, n_tok_blocks, blk,
              w_ref, b_ref, x_ref, gw_ref, gb_ref, o_ref, sb_scr):
    step = pl.program_id(0)
    d = x_ref.shape[1]

    @pl.when(step < n_wsteps)
    def _w_phase():
        for j in range(wblk):
            sb_scr[pl.ds(step * wblk + j, 1), :d] = (
                jnp.sum(w_ref[j], axis=1)[None, :])
            sb_scr[pl.ds(step * wblk + j, 1), d:] = b_ref[pl.ds(j, 1), 0, :]

    @pl.when(step >= n_wsteps)
    def _t_phase():
        x = x_ref[...]
        logits = jax.lax.dot_general(
            x, gw_ref[...], (((1,), (1,)), ((), ())),
            preferred_element_type=jnp.float32,
        ) + gb_ref[...]
        g = jax.nn.softmax(logits, axis=-1)
        e_ids = jax.lax.broadcasted_iota(jnp.int32, g.shape, 1)
        i1 = jnp.argmax(g, axis=-1)
        oh1 = e_ids == i1[:, None]
        i2 = jnp.argmax(jnp.where(oh1, -1.0, g), axis=-1)
        oh2 = e_ids == i2[:, None]
        p = jnp.where(oh1 | oh2, g, 0.0)
        q = jax.lax.dot_general(
            p, sb_scr[...], (((1,), (0,)), ((), ())),
            preferred_element_type=jnp.float32,
            precision=jax.lax.Precision.HIGHEST,
        )
        o_ref[...] = x * q[:, :d] + q[:, d:]


def kernel(x, gate_weight, gate_bias, experts_weights, experts_bias):
    n_tokens, d_in = x.shape
    n_exp, _, d_out = experts_weights.shape
    blk = _TOKEN_BLOCK
    wblk = _WBLK
    n_tok_blocks = n_tokens // blk
    n_wsteps = n_exp // wblk

    body = functools.partial(_moe_body, n_exp, wblk, n_wsteps, n_tok_blocks,
                             blk)
    out = pl.pallas_call(
        body,
        grid=(n_wsteps + n_tok_blocks,),
        in_specs=[
            pl.BlockSpec((wblk, d_in, d_out),
                         lambda i: (jnp.minimum(i, n_wsteps - 1), 0, 0)),
            pl.BlockSpec((wblk, 1, d_out),
                         lambda i: (jnp.minimum(i, n_wsteps - 1), 0, 0)),
            pl.BlockSpec((blk, d_in),
                         lambda i: (jnp.maximum(i - n_wsteps, 0), 0)),
            pl.BlockSpec((n_exp, d_in), lambda i: (0, 0)),
            pl.BlockSpec((1, n_exp), lambda i: (0, 0)),
        ],
        out_specs=pl.BlockSpec((blk, d_out),
                               lambda i: (jnp.maximum(i - n_wsteps, 0), 0)),
        out_shape=jax.ShapeDtypeStruct((n_tokens, d_out), jnp.float32),
        scratch_shapes=[pltpu.VMEM((n_exp, d_in + d_out), jnp.float32)],
    )(experts_weights, experts_bias.reshape(n_exp, 1, d_out), x, gate_weight,
      gate_bias.reshape(1, n_exp))
    return out
